# Initial kernel scaffold; baseline (speedup 1.0000x reference)
import jax
import jax.numpy as jnp
from jax.experimental import pallas as pl


def kernel(p_cat, p_num, p_desc, t_light, t_tol, t_hum, t_water, t_care, t_size, t_climate, W_desc, b_desc, W1, b1, W2, b2, W3, b3):
    raise NotImplementedError("write your pallas kernel here")


# speedup vs baseline: 1.1476x; 1.1476x over previous
import functools

import jax
import jax.numpy as jnp
from jax import lax
from jax.experimental import pallas as pl
from jax.experimental.pallas import tpu as pltpu
from jax.experimental.pallas import tpu_sc as plsc

B = 16384
EMBED = 8
NUM_FEATS = 7
DESC_IN = 1024
DESC_OUT = 64
H1 = 128
H2 = 128
OUT = 64


_NC = 2
_NS = 16
_NW = _NC * _NS
_BPW = B // _NW
_CHUNK = 128
_NCHUNK = _BPW // _CHUNK


def _sc_gather_body(pcat_hbm, t0, t1, t2, t3, t4, t5, t6, out_hbm,
                    idx_v, rows_v, sem):
  tables = (t0, t1, t2, t3, t4, t5, t6)
  wid = lax.axis_index("s") * _NC + lax.axis_index("c")
  base = wid * _BPW
  for i in range(NUM_FEATS):
    pltpu.sync_copy(pcat_hbm.at[i, wid], idx_v)
    for j in range(_NCHUNK):
      pltpu.async_copy(tables[i].at[idx_v.at[j]],
                       rows_v.at[pl.ds(j * _CHUNK, _CHUNK)], sem).wait()
    pltpu.sync_copy(rows_v, out_hbm.at[i, pl.ds(base, _BPW)])


@functools.cache
def _sc_gather():
  return functools.partial(
      pl.kernel,
      out_type=jax.ShapeDtypeStruct((NUM_FEATS, B, EMBED), jnp.float32),
      mesh=plsc.VectorSubcoreMesh(core_axis_name="c", subcore_axis_name="s",
                                  num_cores=_NC),
      scratch_types=[
          pltpu.VMEM((_NCHUNK, _CHUNK), jnp.int32),
          pltpu.VMEM((_BPW, EMBED), jnp.float32),
          pltpu.SemaphoreType.DMA,
      ],
      compiler_params=pltpu.CompilerParams(use_tc_tiling_on_sc=False),
  )(_sc_gather_body)


_BBLK = 1024


def _tc_body(cat_ref, pnum_ref, pdesc_ref, wdesc_ref, bdesc_ref,
             w1a_ref, w1b_ref, w1c_ref, b1_ref, w2_ref, b2_ref,
             w3_ref, b3_ref, out_ref):
  pd = pdesc_ref[...]
  ss = jnp.sum(pd * pd, axis=1, keepdims=True)
  inv = 1.0 / (jnp.sqrt(ss) + 1e-08)
  d0 = jnp.dot(pd, wdesc_ref[...], preferred_element_type=jnp.float32)
  desc = d0 * inv + bdesc_ref[...]
  h = (jnp.dot(pnum_ref[...], w1b_ref[...], preferred_element_type=jnp.float32)
       + jnp.dot(desc, w1c_ref[...], preferred_element_type=jnp.float32)
       + b1_ref[...])
  for i in range(NUM_FEATS):
    h = h + jnp.dot(cat_ref[i], w1a_ref[pl.ds(i * EMBED, EMBED)],
                    preferred_element_type=jnp.float32)
  h = jnp.maximum(h, 0.0)
  h = jnp.maximum(
      jnp.dot(h, w2_ref[...], preferred_element_type=jnp.float32) + b2_ref[...],
      0.0)
  out_ref[...] = (
      jnp.dot(h, w3_ref[...], preferred_element_type=jnp.float32) + b3_ref[...])


def _full(shape):
  return pl.BlockSpec(shape, lambda i: (0,) * len(shape))


def _tc_mlp(cat, p_num, p_desc, W_desc, b_desc, W1a, W1b, W1c, b1, W2, b2,
            W3, b3):
  grid = (B // _BBLK,)
  return pl.pallas_call(
      _tc_body,
      grid=grid,
      in_specs=[
          pl.BlockSpec((NUM_FEATS, _BBLK, EMBED), lambda i: (0, i, 0)),
          pl.BlockSpec((_BBLK, 2), lambda i: (i, 0)),
          pl.BlockSpec((_BBLK, DESC_IN), lambda i: (i, 0)),
          _full((DESC_IN, DESC_OUT)),
          _full((1, DESC_OUT)),
          _full((56, H1)),
          _full((2, H1)),
          _full((DESC_OUT, H1)),
          _full((1, H1)),
          _full((H1, H2)),
          _full((1, H2)),
          _full((H2, OUT)),
          _full((1, OUT)),
      ],
      out_specs=pl.BlockSpec((_BBLK, OUT), lambda i: (i, 0)),
      out_shape=jax.ShapeDtypeStruct((B, OUT), jnp.float32),
  )(cat, p_num, p_desc, W_desc, b_desc, W1a, W1b, W1c, b1, W2, b2, W3, b3)


def kernel(p_cat, p_num, p_desc, t_light, t_tol, t_hum, t_water, t_care,
           t_size, t_climate, W_desc, b_desc, W1, b1, W2, b2, W3, b3):
  pcat_w = p_cat.astype(jnp.int32).T.reshape(NUM_FEATS, _NW, _NCHUNK, _CHUNK)
  cat = _sc_gather()(pcat_w, t_light, t_tol, t_hum, t_water, t_care, t_size,
                     t_climate)
  W1a = W1[:56]
  W1b = W1[56:58]
  W1c = W1[58:]
  return _tc_mlp(cat, p_num, p_desc, W_desc, b_desc.reshape(1, -1), W1a, W1b,
                 W1c, b1.reshape(1, -1), W2, b2.reshape(1, -1), W3,
                 b3.reshape(1, -1))


# speedup vs baseline: 4.0679x; 3.5446x over previous
import functools

import jax
import jax.numpy as jnp
from jax import lax
from jax.experimental import pallas as pl
from jax.experimental.pallas import tpu as pltpu
from jax.experimental.pallas import tpu_sc as plsc

B = 16384
EMBED = 8
NUM_FEATS = 7
DESC_IN = 1024
DESC_OUT = 64
H1 = 128
H2 = 128
OUT = 64


_NC = 2
_NS = 16
_NW = _NC * _NS
_BPW = B // _NW
_CAT = NUM_FEATS * EMBED
_NIDX = _BPW * NUM_FEATS
_NOUT = _BPW * _CAT
_L = 16

_SIZES = (6, 6, 4, 4, 4, 4, 6)
_OFF8 = []
_acc = 0
for _s in _SIZES:
  _OFF8.append(_acc * EMBED)
  _acc += _s
_TAB_LEN = _acc * EMBED


def _sc_gather_body(pcat_hbm, tab_hbm, off_hbm, out_hbm, idx_v, tab_v, off_v,
                    rows_v):
  wid = lax.axis_index("s") * _NC + lax.axis_index("c")
  pltpu.sync_copy(pcat_hbm.at[wid], idx_v)
  pltpu.sync_copy(tab_hbm, tab_v)
  pltpu.sync_copy(off_hbm, off_v)

  def step(k, _):
    n = k * _L + lax.iota(jnp.int32, _L)
    b = jnp.right_shift(n * 37450, 21)
    r = n - b * _CAT
    i = jnp.right_shift(r, 3)
    d = jnp.bitwise_and(r, 7)
    tv = plsc.load_gather(idx_v, [b * NUM_FEATS + i])
    fo = plsc.load_gather(off_v, [i]) + tv * EMBED + d
    val = plsc.load_gather(tab_v, [fo])
    rows_v[pl.ds(k * _L, _L)] = val
    return _

  lax.fori_loop(0, _NOUT // _L, step, 0, unroll=8)
  pltpu.sync_copy(rows_v, out_hbm.at[pl.ds(wid * _NOUT, _NOUT)])


@functools.cache
def _sc_gather():
  return functools.partial(
      pl.kernel,
      out_type=jax.ShapeDtypeStruct((B * _CAT,), jnp.float32),
      mesh=plsc.VectorSubcoreMesh(core_axis_name="c", subcore_axis_name="s",
                                  num_cores=_NC),
      scratch_types=[
          pltpu.VMEM((_NIDX,), jnp.int32),
          pltpu.VMEM((_TAB_LEN,), jnp.float32),
          pltpu.VMEM((EMBED,), jnp.int32),
          pltpu.VMEM((_NOUT,), jnp.float32),
      ],
      compiler_params=pltpu.CompilerParams(use_tc_tiling_on_sc=False,
                                           needs_layout_passes=False),
  )(_sc_gather_body)


_BBLK = 1024


def _tc_body(cat_ref, pnum_ref, pdesc_ref, wdesc_ref, bdesc_ref,
             w1a_ref, w1b_ref, w1c_ref, b1_ref, w2_ref, b2_ref,
             w3_ref, b3_ref, out_ref):
  pd = pdesc_ref[...]
  ss = jnp.sum(pd * pd, axis=1, keepdims=True)
  inv = 1.0 / (jnp.sqrt(ss) + 1e-08)
  d0 = jnp.dot(pd, wdesc_ref[...], preferred_element_type=jnp.float32)
  desc = d0 * inv + bdesc_ref[...]
  h = (jnp.dot(cat_ref[...], w1a_ref[...], preferred_element_type=jnp.float32)
       + jnp.dot(pnum_ref[...], w1b_ref[...], preferred_element_type=jnp.float32)
       + jnp.dot(desc, w1c_ref[...], preferred_element_type=jnp.float32)
       + b1_ref[...])
  h = jnp.maximum(h, 0.0)
  h = jnp.maximum(
      jnp.dot(h, w2_ref[...], preferred_element_type=jnp.float32) + b2_ref[...],
      0.0)
  out_ref[...] = (
      jnp.dot(h, w3_ref[...], preferred_element_type=jnp.float32) + b3_ref[...])


def _full(shape):
  return pl.BlockSpec(shape, lambda i: (0,) * len(shape))


def _tc_mlp(cat, p_num, p_desc, W_desc, b_desc, W1a, W1b, W1c, b1, W2, b2,
            W3, b3):
  grid = (B // _BBLK,)
  return pl.pallas_call(
      _tc_body,
      grid=grid,
      in_specs=[
          pl.BlockSpec((_BBLK, _CAT), lambda i: (i, 0)),
          pl.BlockSpec((_BBLK, 2), lambda i: (i, 0)),
          pl.BlockSpec((_BBLK, DESC_IN), lambda i: (i, 0)),
          _full((DESC_IN, DESC_OUT)),
          _full((1, DESC_OUT)),
          _full((_CAT, H1)),
          _full((2, H1)),
          _full((DESC_OUT, H1)),
          _full((1, H1)),
          _full((H1, H2)),
          _full((1, H2)),
          _full((H2, OUT)),
          _full((1, OUT)),
      ],
      out_specs=pl.BlockSpec((_BBLK, OUT), lambda i: (i, 0)),
      out_shape=jax.ShapeDtypeStruct((B, OUT), jnp.float32),
  )(cat, p_num, p_desc, W_desc, b_desc, W1a, W1b, W1c, b1, W2, b2, W3, b3)


def kernel(p_cat, p_num, p_desc, t_light, t_tol, t_hum, t_water, t_care,
           t_size, t_climate, W_desc, b_desc, W1, b1, W2, b2, W3, b3):
  pcat_w = p_cat.astype(jnp.int32).reshape(_NW, _NIDX)
  tab_flat = jnp.concatenate(
      [t.reshape(-1) for t in
       (t_light, t_tol, t_hum, t_water, t_care, t_size, t_climate)])
  off8 = jnp.array(_OFF8 + [0], dtype=jnp.int32)
  cat_flat = _sc_gather()(pcat_w, tab_flat, off8)
  cat = cat_flat.reshape(B, _CAT)
  W1a = W1[:56]
  W1b = W1[56:58]
  W1c = W1[58:]
  return _tc_mlp(cat, p_num, p_desc, W_desc, b_desc.reshape(1, -1), W1a, W1b,
                 W1c, b1.reshape(1, -1), W2, b2.reshape(1, -1), W3,
                 b3.reshape(1, -1))


# speedup vs baseline: 4.1807x; 1.0277x over previous
import functools

import jax
import jax.numpy as jnp
from jax import lax
from jax.experimental import pallas as pl
from jax.experimental.pallas import tpu as pltpu
from jax.experimental.pallas import tpu_sc as plsc

B = 16384
EMBED = 8
NUM_FEATS = 7
DESC_IN = 1024
DESC_OUT = 64
H1 = 128
H2 = 128
OUT = 64


_NC = 2
_NS = 16
_NW = _NC * _NS
_BPW = B // _NW
_CAT = NUM_FEATS * EMBED
_NIDX = _BPW * NUM_FEATS
_NOUT = _BPW * _CAT
_L = 16

_SIZES = (6, 6, 4, 4, 4, 4, 6)
_OFF8 = []
_acc = 0
for _s in _SIZES:
  _OFF8.append(_acc * EMBED)
  _acc += _s
_TAB_LEN = _acc * EMBED


def _sc_gather_body(pcat_hbm, tab_hbm, off_hbm, out_hbm, idx_v, tab_v, off_v,
                    rows_v):
  wid = lax.axis_index("s") * _NC + lax.axis_index("c")
  pltpu.sync_copy(pcat_hbm.at[wid], idx_v)
  pltpu.sync_copy(tab_hbm, tab_v)
  pltpu.sync_copy(off_hbm, off_v)

  def step(k, _):
    n = k * _L + lax.iota(jnp.int32, _L)
    b = jnp.right_shift(n * 37450, 21)
    r = n - b * _CAT
    i = jnp.right_shift(r, 3)
    d = jnp.bitwise_and(r, 7)
    tv = plsc.load_gather(idx_v, [b * NUM_FEATS + i])
    fo = plsc.load_gather(off_v, [i]) + tv * EMBED + d
    val = plsc.load_gather(tab_v, [fo])
    rows_v[pl.ds(k * _L, _L)] = val
    return _

  lax.fori_loop(0, _NOUT // _L, step, 0, unroll=8)
  pltpu.sync_copy(rows_v, out_hbm.at[pl.ds(wid * _NOUT, _NOUT)])


@functools.cache
def _sc_gather():
  return functools.partial(
      pl.kernel,
      out_type=jax.ShapeDtypeStruct((B * _CAT,), jnp.float32),
      mesh=plsc.VectorSubcoreMesh(core_axis_name="c", subcore_axis_name="s",
                                  num_cores=_NC),
      scratch_types=[
          pltpu.VMEM((_NIDX,), jnp.int32),
          pltpu.VMEM((_TAB_LEN,), jnp.float32),
          pltpu.VMEM((EMBED,), jnp.int32),
          pltpu.VMEM((_NOUT,), jnp.float32),
      ],
      compiler_params=pltpu.CompilerParams(use_tc_tiling_on_sc=False,
                                           needs_layout_passes=False),
  )(_sc_gather_body)


_BBLK = 1024


def _tc_desc_body(pdesc_ref, wdesc_ref, bdesc_ref, out_ref):
  pd = pdesc_ref[...]
  ss = jnp.sum(pd * pd, axis=1, keepdims=True)
  inv = 1.0 / (jnp.sqrt(ss) + 1e-08)
  d0 = jnp.dot(pd, wdesc_ref[...], preferred_element_type=jnp.float32)
  out_ref[...] = d0 * inv + bdesc_ref[...]


def _tc_mlp_body(cat_ref, pnum_ref, desc_ref, w1a_ref, w1b_ref, w1c_ref,
                 b1_ref, w2_ref, b2_ref, w3_ref, b3_ref, out_ref):
  h = (jnp.dot(cat_ref[...], w1a_ref[...], preferred_element_type=jnp.float32)
       + jnp.dot(pnum_ref[...], w1b_ref[...], preferred_element_type=jnp.float32)
       + jnp.dot(desc_ref[...], w1c_ref[...], preferred_element_type=jnp.float32)
       + b1_ref[...])
  h = jnp.maximum(h, 0.0)
  h = jnp.maximum(
      jnp.dot(h, w2_ref[...], preferred_element_type=jnp.float32) + b2_ref[...],
      0.0)
  out_ref[...] = (
      jnp.dot(h, w3_ref[...], preferred_element_type=jnp.float32) + b3_ref[...])


def _full(shape):
  return pl.BlockSpec(shape, lambda i: (0,) * len(shape))


def _tc_desc(p_desc, W_desc, b_desc):
  return pl.pallas_call(
      _tc_desc_body,
      grid=(B // _BBLK,),
      in_specs=[
          pl.BlockSpec((_BBLK, DESC_IN), lambda i: (i, 0)),
          _full((DESC_IN, DESC_OUT)),
          _full((1, DESC_OUT)),
      ],
      out_specs=pl.BlockSpec((_BBLK, DESC_OUT), lambda i: (i, 0)),
      out_shape=jax.ShapeDtypeStruct((B, DESC_OUT), jnp.float32),
  )(p_desc, W_desc, b_desc)


def _tc_mlp(cat, p_num, desc, W1a, W1b, W1c, b1, W2, b2, W3, b3):
  return pl.pallas_call(
      _tc_mlp_body,
      grid=(B // _BBLK,),
      in_specs=[
          pl.BlockSpec((_BBLK, _CAT), lambda i: (i, 0)),
          pl.BlockSpec((_BBLK, 2), lambda i: (i, 0)),
          pl.BlockSpec((_BBLK, DESC_OUT), lambda i: (i, 0)),
          _full((_CAT, H1)),
          _full((2, H1)),
          _full((DESC_OUT, H1)),
          _full((1, H1)),
          _full((H1, H2)),
          _full((1, H2)),
          _full((H2, OUT)),
          _full((1, OUT)),
      ],
      out_specs=pl.BlockSpec((_BBLK, OUT), lambda i: (i, 0)),
      out_shape=jax.ShapeDtypeStruct((B, OUT), jnp.float32),
  )(cat, p_num, desc, W1a, W1b, W1c, b1, W2, b2, W3, b3)


def kernel(p_cat, p_num, p_desc, t_light, t_tol, t_hum, t_water, t_care,
           t_size, t_climate, W_desc, b_desc, W1, b1, W2, b2, W3, b3):
  pcat_w = p_cat.astype(jnp.int32).reshape(_NW, _NIDX)
  tab_flat = jnp.concatenate(
      [t.reshape(-1) for t in
       (t_light, t_tol, t_hum, t_water, t_care, t_size, t_climate)])
  off8 = jnp.array(_OFF8 + [0], dtype=jnp.int32)
  cat_flat = _sc_gather()(pcat_w, tab_flat, off8)
  cat = cat_flat.reshape(B, _CAT)
  desc = _tc_desc(p_desc, W_desc, b_desc.reshape(1, -1))
  W1a = W1[:56]
  W1b = W1[56:58]
  W1c = W1[58:]
  return _tc_mlp(cat, p_num, desc, W1a, W1b, W1c, b1.reshape(1, -1), W2,
                 b2.reshape(1, -1), W3, b3.reshape(1, -1))
